# baseline (speedup 1.0000x reference)
import functools
import math

import jax
import jax.numpy as jnp
from jax import lax
from jax.experimental import pallas as pl
from jax.experimental.pallas import tpu as pltpu
from jax.experimental.pallas import tpu_sc as plsc

LANES = 16


@functools.cache
def _build(B, T, V, D, PE_LEN):
    NC, NS = 2, 16
    NW = NC * NS
    PPW = T // NW
    P = 32
    NCH = PPW // P
    VPR = D // LANES
    SCALE = math.sqrt(float(D))

    mesh = plsc.VectorSubcoreMesh(core_axis_name="c", subcore_axis_name="s")

    @functools.partial(
        pl.kernel,
        out_type=jax.ShapeDtypeStruct((B * T, D), jnp.float32),
        mesh=mesh,
        scratch_types=[
            pltpu.VMEM((P,), jnp.int32),
            pltpu.VMEM((P, D), jnp.float32),
            pltpu.VMEM((P, D), jnp.float32),
            pltpu.SemaphoreType.DMA,
        ],
    )
    def emb_kernel(x_ref, table_ref, pe_ref, out_ref, idx_v, pe_v, tok_v, sem):
        wid = lax.axis_index("s") * NC + lax.axis_index("c")
        for ch in range(NCH):
            pos0 = wid * PPW + ch * P
            pltpu.sync_copy(pe_ref.at[pl.ds(pos0, P)], pe_v)
            for b in range(B):
                row0 = b * T + pos0
                pltpu.sync_copy(x_ref.at[pl.ds(row0, P)], idx_v)
                pltpu.async_copy(table_ref.at[idx_v], tok_v, sem).wait()

                def fma_row(r, _):
                    for cv in range(VPR):
                        sl = pl.ds(cv * LANES, LANES)
                        tok_v[r, sl] = tok_v[r, sl] * SCALE + pe_v[r, sl]
                    return 0

                lax.fori_loop(0, P, fma_row, 0)
                pltpu.sync_copy(tok_v, out_ref.at[pl.ds(row0, P)])

    return emb_kernel


def kernel(x, table, pe):
    B, T = x.shape
    V, D = table.shape
    out = _build(B, T, V, D, pe.shape[0])(
        x.reshape(B * T).astype(jnp.int32), table, pe
    )
    return out.reshape(B, T, D)


# speedup vs baseline: 1.5738x; 1.5738x over previous
import functools
import math

import jax
import jax.numpy as jnp
from jax import lax
from jax.experimental import pallas as pl
from jax.experimental.pallas import tpu as pltpu
from jax.experimental.pallas import tpu_sc as plsc

LANES = 16


@functools.cache
def _build(B, T, V, D, PE_LEN):
    NC, NS = 2, 16
    NW = NC * NS
    PPW = T // NW
    P = 32
    NCH = PPW // P
    VPR = D // LANES
    NTOK = 3
    SCALE = math.sqrt(float(D))
    UNITS = [(ch, b) for ch in range(NCH) for b in range(B)]
    NU = len(UNITS)

    mesh = plsc.VectorSubcoreMesh(core_axis_name="c", subcore_axis_name="s")

    @functools.partial(
        pl.kernel,
        out_type=jax.ShapeDtypeStruct((B * T, D), jnp.float32),
        mesh=mesh,
        scratch_types=[
            pltpu.VMEM((B, PPW), jnp.int32),
            [pltpu.VMEM((P, D), jnp.float32) for _ in range(2)],
            [pltpu.VMEM((P, D), jnp.float32) for _ in range(NTOK)],
            pltpu.SemaphoreType.DMA,
            pltpu.SemaphoreType.DMA,
            pltpu.SemaphoreType.DMA,
            pltpu.SemaphoreType.DMA,
        ],
    )
    def emb_kernel(x_ref, table_ref, pe_ref, out_ref,
                   idx_v, pe_v, tok_v, isem, psem, gsem, wsem):
        wid = lax.axis_index("s") * NC + lax.axis_index("c")
        pos_base = wid * PPW

        idx_copies = [
            pltpu.async_copy(x_ref.at[pl.ds(b * T + pos_base, PPW)],
                             idx_v.at[b], isem)
            for b in range(B)
        ]

        def pe_fetch(ch):
            return pltpu.async_copy(
                pe_ref.at[pl.ds(pos_base + ch * P, P)], pe_v[ch % 2], psem)

        def gather(u):
            ch, b = UNITS[u]
            return pltpu.async_copy(
                table_ref.at[idx_v.at[b, pl.ds(ch * P, P)]],
                tok_v[u % NTOK], gsem)

        def write(u):
            ch, b = UNITS[u]
            return pltpu.async_copy(
                tok_v[u % NTOK],
                out_ref.at[pl.ds(b * T + pos_base + ch * P, P)], wsem)

        pe_pending = [pe_fetch(0)]
        for c in idx_copies:
            c.wait()
        gather_pending = [gather(0)]
        write_pending = []

        for u in range(NU):
            nxt = u + 1
            if nxt < NU:
                if nxt % B == 0:
                    pe_pending.append(pe_fetch(nxt // B))
                if nxt >= NTOK:
                    write_pending.pop(0).wait()
                gather_pending.append(gather(nxt))
            if u % B == 0:
                pe_pending.pop(0).wait()
            gather_pending.pop(0).wait()

            pe_buf = pe_v[(u // B) % 2]
            tok_buf = tok_v[u % NTOK]

            def fma_row(r, _):
                for cv in range(VPR):
                    sl = pl.ds(cv * LANES, LANES)
                    tok_buf[r, sl] = tok_buf[r, sl] * SCALE + pe_buf[r, sl]
                return 0

            lax.fori_loop(0, P, fma_row, 0)
            write_pending.append(write(u))

        for w in write_pending:
            w.wait()

    return emb_kernel


def kernel(x, table, pe):
    B, T = x.shape
    V, D = table.shape
    out = _build(B, T, V, D, pe.shape[0])(
        x.reshape(B * T).astype(jnp.int32), table, pe
    )
    return out.reshape(B, T, D)
